# baseline (speedup 1.0000x reference)
import jax
import jax.numpy as jnp
from jax import lax
from jax.experimental import pallas as pl
from jax.experimental.pallas import tpu as pltpu
from jax.experimental.pallas import tpu_sc as plsc

N_TOKENS = 32768
N_EXPERTS = 64
K = 8
L = 16
NC = 2
NS = 16
NW = NC * NS
ROWS_PER_W = N_TOKENS // NW


def _topk_body(logits_hbm, w_hbm, ids_hbm, logits_v, w_v, ids_v):
    wid = lax.axis_index("s") * NC + lax.axis_index("c")
    row_base = wid * ROWS_PER_W
    pltpu.sync_copy(logits_hbm.at[pl.ds(row_base, ROWS_PER_W), :], logits_v)

    iota = lax.iota(jnp.int32, L)
    lane_lt8 = iota < K
    col8 = jnp.where(lane_lt8, iota, 0)
    group_ids = [iota + g * L for g in range(4)]

    def merge(av, ai, bv, bi):
        cv = jnp.where(lane_lt8, av, lax.rev(bv, (0,)))
        ci = jnp.where(lane_lt8, ai, lax.rev(bi, (0,)))
        return plsc.sort_key_val(cv, ci, descending=True)

    def one_row(r, _):
        sv, si = [], []
        for g in range(4):
            v = logits_v[r, pl.ds(g * L, L)]
            k, x = plsc.sort_key_val(v, group_ids[g], descending=True)
            sv.append(k)
            si.append(x)
        mv0, mi0 = merge(sv[0], si[0], sv[1], si[1])
        mv1, mi1 = merge(sv[2], si[2], sv[3], si[3])
        fv, fi = merge(mv0, mi0, mv1, mi1)
        e = jnp.exp(fv - jnp.max(fv))
        denom = jnp.sum(jnp.where(lane_lt8, e, 0.0))
        w = e / denom
        row = jnp.full((L,), r, dtype=jnp.int32)
        plsc.store_scatter(w_v, [row, col8], w, mask=lane_lt8)
        plsc.store_scatter(ids_v, [row, col8], fi, mask=lane_lt8)
        return _

    lax.fori_loop(0, ROWS_PER_W, one_row, 0)

    pltpu.sync_copy(w_v, w_hbm.at[pl.ds(row_base, ROWS_PER_W), :])
    pltpu.sync_copy(ids_v, ids_hbm.at[pl.ds(row_base, ROWS_PER_W), :])


def kernel(hidden_states, router_logits):
    del hidden_states
    fn = pl.kernel(
        _topk_body,
        out_type=(
            jax.ShapeDtypeStruct((N_TOKENS, K), jnp.float32),
            jax.ShapeDtypeStruct((N_TOKENS, K), jnp.int32),
        ),
        mesh=plsc.VectorSubcoreMesh(core_axis_name="c", subcore_axis_name="s"),
        compiler_params=pltpu.CompilerParams(
            needs_layout_passes=False, use_tc_tiling_on_sc=False),
        scratch_types=[
            pltpu.VMEM((ROWS_PER_W, N_EXPERTS), jnp.float32),
            pltpu.VMEM((ROWS_PER_W, K), jnp.float32),
            pltpu.VMEM((ROWS_PER_W, K), jnp.int32),
        ],
    )
    topk_weights, topk_ids = fn(router_logits)
    return topk_weights, topk_ids, router_logits


# speedup vs baseline: 1.5221x; 1.5221x over previous
import jax
import jax.numpy as jnp
from jax import lax
from jax.experimental import pallas as pl
from jax.experimental.pallas import tpu as pltpu
from jax.experimental.pallas import tpu_sc as plsc

N_TOKENS = 32768
N_EXPERTS = 64
K = 8
L = 16
NC = 2
NS = 16
NW = NC * NS
ROWS_PER_W = N_TOKENS // NW


def _topk_body(logits_hbm, w_hbm, ids_hbm, logits_v, w_v, ids_v):
    wid = lax.axis_index("s") * NC + lax.axis_index("c")
    in_base = wid * (ROWS_PER_W * N_EXPERTS)
    pltpu.sync_copy(logits_hbm.at[pl.ds(in_base, ROWS_PER_W * N_EXPERTS)],
                    logits_v)

    iota = lax.iota(jnp.int32, L)
    lane_lt8 = iota < K
    gidx = jnp.maximum(iota - K, 0)
    group_ids = [iota + g * L for g in range(4)]

    def merge(av, ai, bv, bi):
        cv = jnp.where(lane_lt8, av, lax.rev(bv, (0,)))
        ci = jnp.where(lane_lt8, ai, lax.rev(bi, (0,)))
        return plsc.sort_key_val(cv, ci, descending=True)

    def one_row(off):
        sv, si = [], []
        for g in range(4):
            v = logits_v[pl.ds(off + g * L, L)]
            k, x = plsc.sort_key_val(v, group_ids[g], descending=True)
            sv.append(k)
            si.append(x)
        mv0, mi0 = merge(sv[0], si[0], sv[1], si[1])
        mv1, mi1 = merge(sv[2], si[2], sv[3], si[3])
        fv, fi = merge(mv0, mi0, mv1, mi1)
        e = jnp.exp(fv - jnp.max(fv))
        denom = jnp.sum(jnp.where(lane_lt8, e, 0.0))
        return e / denom, fi

    @plsc.parallel_loop(0, ROWS_PER_W // 2, unroll=4)
    def two_rows(j):
        wa, ia = one_row(2 * j * N_EXPERTS)
        wb, ib = one_row((2 * j + 1) * N_EXPERTS)
        wb8 = wb.at[gidx].get(mode="promise_in_bounds")
        ib8 = ib.at[gidx].get(mode="promise_in_bounds")
        w_v[pl.ds(j * L, L)] = jnp.where(lane_lt8, wa, wb8)
        ids_v[pl.ds(j * L, L)] = jnp.where(lane_lt8, ia, ib8)

    out_base = wid * (ROWS_PER_W * K)
    pltpu.sync_copy(w_v, w_hbm.at[pl.ds(out_base, ROWS_PER_W * K)])
    pltpu.sync_copy(ids_v, ids_hbm.at[pl.ds(out_base, ROWS_PER_W * K)])


def kernel(hidden_states, router_logits):
    del hidden_states
    fn = pl.kernel(
        _topk_body,
        out_type=(
            jax.ShapeDtypeStruct((N_TOKENS * K,), jnp.float32),
            jax.ShapeDtypeStruct((N_TOKENS * K,), jnp.int32),
        ),
        mesh=plsc.VectorSubcoreMesh(core_axis_name="c", subcore_axis_name="s"),
        compiler_params=pltpu.CompilerParams(needs_layout_passes=False),
        scratch_types=[
            pltpu.VMEM((ROWS_PER_W * N_EXPERTS,), jnp.float32),
            pltpu.VMEM((ROWS_PER_W * K,), jnp.float32),
            pltpu.VMEM((ROWS_PER_W * K,), jnp.int32),
        ],
    )
    w_flat, ids_flat = fn(router_logits.reshape(-1))
    return (w_flat.reshape(N_TOKENS, K),
            ids_flat.reshape(N_TOKENS, K),
            router_logits)
